# baseline (speedup 1.0000x reference)
import jax
import jax.numpy as jnp
from jax.experimental import pallas as pl
from jax.experimental.pallas import tpu as pltpu

B = 32768
BLK = 2048
NE = 4
D = 64


def _first_eq_rows(rows, m):
    e = [(r == m).astype(jnp.float32) for r in rows]
    f0 = e[0]
    f1 = e[1] * (1.0 - f0)
    f2 = e[2] * (1.0 - f0) * (1.0 - e[1])
    f3 = e[3] * (1.0 - f0) * (1.0 - e[1]) * (1.0 - e[2])
    return [f0, f1, f2, f3]


def _body(t_ref, aux_ref, wr_ref, brc_ref, wf_ref, ws_ref, wk_ref, ww_ref,
          emb_ref, w_ref, mask_ref, wblk_ref):
    @pl.when(pl.program_id(0) == 0)
    def _build():
        z = jnp.zeros((16, D), dtype=jnp.float32)
        rows = []
        for i, ref in enumerate((wf_ref, ws_ref, wk_ref, ww_ref)):
            parts = [z] * NE
            parts[i] = ref[...]
            rows.append(jnp.concatenate(parts, axis=1))
        wblk_ref[...] = jnp.concatenate(rows, axis=0)

    t = t_ref[...]
    auxt = aux_ref[...].T

    rin = jnp.concatenate([t, auxt], axis=0)
    logits = jax.lax.dot_general(
        wr_ref[...], rin, (((0,), (0,)), ((), ())),
        preferred_element_type=jnp.float32)
    logits = logits + brc_ref[...]
    m = jnp.max(logits, axis=0, keepdims=True)
    e = jnp.exp(logits - m)
    s = jnp.sum(e, axis=0, keepdims=True)
    w = e / s

    wr = [w[k:k + 1] for k in range(4)]
    m1 = jnp.max(w, axis=0, keepdims=True)
    f1 = _first_eq_rows(wr, m1)
    wmr = [wr[k] - 2.0 * f1[k] for k in range(4)]
    wm = jnp.concatenate(wmr, axis=0)
    m2 = jnp.max(wm, axis=0, keepdims=True)
    f2 = _first_eq_rows(wmr, m2)
    sel = [jnp.minimum(f1[k] + f2[k], 1.0) for k in range(4)]
    disp = jnp.concatenate([wr[k] * sel[k] for k in range(4)], axis=0)

    w_ref[...] = w.T
    mask_ref[...] = jnp.concatenate(sel, axis=0).T

    i8 = jax.lax.broadcasted_iota(jnp.int32, (8, 1), 0).astype(jnp.float32)
    i16 = jax.lax.broadcasted_iota(jnp.int32, (16, 1), 0).astype(jnp.float32)
    u = (i8 + 1.0) * t
    four = jnp.concatenate([jnp.sin(u), jnp.cos(u)], axis=0)

    grid = i16 * (1.0 / 15.0)
    us = (t - grid) * 8.0
    bsp = jnp.maximum(1.0 - jnp.abs(us), 0.0)
    spl = bsp * bsp * bsp

    dg = t - grid
    rk = jnp.exp(-10.0 * dg * dg)

    quo = jnp.floor(i16 * 0.25)
    tr_col = (i16 - 4.0 * quo) * (1.0 / 3.0)
    inv_sc = jnp.exp2(1.0 - quo)
    uw = (t - tr_col) * inv_sc
    uw2 = uw * uw
    wav = (1.0 - uw2) * jnp.exp(-0.5 * uw2)

    sb = jnp.concatenate([four * disp[0:1], spl * disp[1:2],
                          rk * disp[2:3], wav * disp[3:4]], axis=0)
    emb_ref[...] = jax.lax.dot_general(
        sb, wblk_ref[...], (((0,), (0,)), ((), ())),
        preferred_element_type=jnp.float32)


def kernel(timestamp_input, auxiliary_features, Wr, br,
           W_fourier, W_spline, W_rkhs, W_wavelet):
    t_row = timestamp_input.reshape(1, B)
    brc = br.reshape(NE, 1)

    grid = (B // BLK,)
    emb, w_n, mask_n = pl.pallas_call(
        _body,
        grid=grid,
        in_specs=[
            pl.BlockSpec((1, BLK), lambda i: (0, i)),
            pl.BlockSpec((BLK, 16), lambda i: (i, 0)),
            pl.BlockSpec((17, NE), lambda i: (0, 0)),
            pl.BlockSpec((NE, 1), lambda i: (0, 0)),
            pl.BlockSpec((16, D), lambda i: (0, 0)),
            pl.BlockSpec((16, D), lambda i: (0, 0)),
            pl.BlockSpec((16, D), lambda i: (0, 0)),
            pl.BlockSpec((16, D), lambda i: (0, 0)),
        ],
        out_specs=[
            pl.BlockSpec((BLK, 4 * D), lambda i: (i, 0)),
            pl.BlockSpec((BLK, NE), lambda i: (i, 0)),
            pl.BlockSpec((BLK, NE), lambda i: (i, 0)),
        ],
        out_shape=[
            jax.ShapeDtypeStruct((B, 4 * D), jnp.float32),
            jax.ShapeDtypeStruct((B, NE), jnp.float32),
            jax.ShapeDtypeStruct((B, NE), jnp.float32),
        ],
        scratch_shapes=[pltpu.VMEM((4 * 16, 4 * D), jnp.float32)],
    )(t_row, auxiliary_features, Wr, brc,
      W_fourier, W_spline, W_rkhs, W_wavelet)

    return emb, w_n, mask_n.astype(jnp.bool_)


# speedup vs baseline: 1.6835x; 1.6835x over previous
import jax
import jax.numpy as jnp
from jax.experimental import pallas as pl
from jax.experimental.pallas import tpu as pltpu

B = 32768
BLK = 2048
NE = 4
D = 64


def _first_eq_rows(rows, m):
    e = [(r == m).astype(jnp.float32) for r in rows]
    f0 = e[0]
    f1 = e[1] * (1.0 - f0)
    f2 = e[2] * (1.0 - f0) * (1.0 - e[1])
    f3 = e[3] * (1.0 - f0) * (1.0 - e[1]) * (1.0 - e[2])
    return [f0, f1, f2, f3]


def _body(t_ref, aux_ref, wr_ref, brc_ref, wf_ref, ws_ref, wk_ref, ww_ref,
          emb_ref, w_ref, mask_ref, wblk_ref):
    @pl.when(pl.program_id(0) == 0)
    def _build():
        z = jnp.zeros((16, D), dtype=jnp.float32)
        rows = []
        for i, ref in enumerate((wf_ref, ws_ref, wk_ref, ww_ref)):
            parts = [z] * NE
            parts[i] = ref[...]
            rows.append(jnp.concatenate(parts, axis=1))
        wblk_ref[...] = jnp.concatenate(rows, axis=0)

    t = t_ref[...]
    auxt = aux_ref[...].T

    rin = jnp.concatenate([t, auxt], axis=0)
    logits = jax.lax.dot_general(
        wr_ref[...], rin, (((0,), (0,)), ((), ())),
        preferred_element_type=jnp.float32)
    logits = logits + brc_ref[...]
    m = jnp.max(logits, axis=0, keepdims=True)
    e = jnp.exp(logits - m)
    s = jnp.sum(e, axis=0, keepdims=True)
    w = e / s

    wr = [w[k:k + 1] for k in range(4)]
    m1 = jnp.max(w, axis=0, keepdims=True)
    f1 = _first_eq_rows(wr, m1)
    wmr = [wr[k] - 2.0 * f1[k] for k in range(4)]
    wm = jnp.concatenate(wmr, axis=0)
    m2 = jnp.max(wm, axis=0, keepdims=True)
    f2 = _first_eq_rows(wmr, m2)
    sel = [jnp.minimum(f1[k] + f2[k], 1.0) for k in range(4)]
    disp = jnp.concatenate([wr[k] * sel[k] for k in range(4)], axis=0)

    w_ref[...] = w
    mask_ref[...] = jnp.concatenate(sel, axis=0)

    i8 = jax.lax.broadcasted_iota(jnp.int32, (8, 1), 0).astype(jnp.float32)
    i16 = jax.lax.broadcasted_iota(jnp.int32, (16, 1), 0).astype(jnp.float32)
    u = (i8 + 1.0) * t
    four = jnp.concatenate([jnp.sin(u), jnp.cos(u)], axis=0)

    grid = i16 * (1.0 / 15.0)
    us = (t - grid) * 8.0
    bsp = jnp.maximum(1.0 - jnp.abs(us), 0.0)
    spl = bsp * bsp * bsp

    dg = t - grid
    rk = jnp.exp(-10.0 * dg * dg)

    quo = jnp.floor(i16 * 0.25)
    tr_col = (i16 - 4.0 * quo) * (1.0 / 3.0)
    inv_sc = jnp.exp2(1.0 - quo)
    uw = (t - tr_col) * inv_sc
    uw2 = uw * uw
    wav = (1.0 - uw2) * jnp.exp(-0.5 * uw2)

    sb = jnp.concatenate([four * disp[0:1], spl * disp[1:2],
                          rk * disp[2:3], wav * disp[3:4]], axis=0)
    emb_ref[...] = jax.lax.dot_general(
        sb, wblk_ref[...], (((0,), (0,)), ((), ())),
        preferred_element_type=jnp.float32)


def kernel(timestamp_input, auxiliary_features, Wr, br,
           W_fourier, W_spline, W_rkhs, W_wavelet):
    t_row = timestamp_input.reshape(1, B)
    brc = br.reshape(NE, 1)

    grid = (B // BLK,)
    emb, w_n, mask_n = pl.pallas_call(
        _body,
        grid=grid,
        in_specs=[
            pl.BlockSpec((1, BLK), lambda i: (0, i)),
            pl.BlockSpec((BLK, 16), lambda i: (i, 0)),
            pl.BlockSpec((17, NE), lambda i: (0, 0)),
            pl.BlockSpec((NE, 1), lambda i: (0, 0)),
            pl.BlockSpec((16, D), lambda i: (0, 0)),
            pl.BlockSpec((16, D), lambda i: (0, 0)),
            pl.BlockSpec((16, D), lambda i: (0, 0)),
            pl.BlockSpec((16, D), lambda i: (0, 0)),
        ],
        out_specs=[
            pl.BlockSpec((BLK, 4 * D), lambda i: (i, 0)),
            pl.BlockSpec((NE, BLK), lambda i: (0, i)),
            pl.BlockSpec((NE, BLK), lambda i: (0, i)),
        ],
        out_shape=[
            jax.ShapeDtypeStruct((B, 4 * D), jnp.float32),
            jax.ShapeDtypeStruct((NE, B), jnp.float32),
            jax.ShapeDtypeStruct((NE, B), jnp.float32),
        ],
        scratch_shapes=[pltpu.VMEM((4 * 16, 4 * D), jnp.float32)],
    )(t_row, auxiliary_features, Wr, brc,
      W_fourier, W_spline, W_rkhs, W_wavelet)

    return emb, w_n.T, mask_n.T.astype(jnp.bool_)


# speedup vs baseline: 2.5709x; 1.5272x over previous
import jax
import jax.numpy as jnp
from jax.experimental import pallas as pl

B = 32768
BLK = 4096
NE = 4
D = 64


def _first_eq_rows(rows, m):
    e = [(r == m).astype(jnp.float32) for r in rows]
    f0 = e[0]
    f1 = e[1] * (1.0 - f0)
    f2 = e[2] * (1.0 - f0) * (1.0 - e[1])
    f3 = e[3] * (1.0 - f0) * (1.0 - e[1]) * (1.0 - e[2])
    return [f0, f1, f2, f3]


def _body(t_ref, auxt_ref, wrt_ref, brc_ref, wblk_ref,
          emb_ref, w_ref, mask_ref):
    t = t_ref[...]
    auxt = auxt_ref[...]

    rin = jnp.concatenate([t, auxt], axis=0)
    logits = jnp.dot(wrt_ref[...], rin,
                     preferred_element_type=jnp.float32)
    logits = logits + brc_ref[...]
    m = jnp.max(logits, axis=0, keepdims=True)
    e = jnp.exp(logits - m)
    s = jnp.sum(e, axis=0, keepdims=True)
    w = e / s

    wr = [w[k:k + 1] for k in range(4)]
    m1 = jnp.max(w, axis=0, keepdims=True)
    f1 = _first_eq_rows(wr, m1)
    wmr = [wr[k] - 2.0 * f1[k] for k in range(4)]
    wm = jnp.concatenate(wmr, axis=0)
    m2 = jnp.max(wm, axis=0, keepdims=True)
    f2 = _first_eq_rows(wmr, m2)
    sel = [jnp.minimum(f1[k] + f2[k], 1.0) for k in range(4)]
    disp = jnp.concatenate([wr[k] * sel[k] for k in range(4)], axis=0)

    w_ref[...] = w
    mask_ref[...] = jnp.concatenate(sel, axis=0)

    i8 = jax.lax.broadcasted_iota(jnp.int32, (8, 1), 0).astype(jnp.float32)
    i16 = jax.lax.broadcasted_iota(jnp.int32, (16, 1), 0).astype(jnp.float32)
    u = (i8 + 1.0) * t
    four = jnp.concatenate([jnp.sin(u), jnp.cos(u)], axis=0)

    grid = i16 * (1.0 / 15.0)
    us = (t - grid) * 8.0
    bsp = jnp.maximum(1.0 - jnp.abs(us), 0.0)
    spl = bsp * bsp * bsp

    dg = t - grid
    rk = jnp.exp(-10.0 * dg * dg)

    quo = jnp.floor(i16 * 0.25)
    tr_col = (i16 - 4.0 * quo) * (1.0 / 3.0)
    inv_sc = jnp.exp2(1.0 - quo)
    uw = (t - tr_col) * inv_sc
    uw2 = uw * uw
    wav = (1.0 - uw2) * jnp.exp(-0.5 * uw2)

    sb = jnp.concatenate([four * disp[0:1], spl * disp[1:2],
                          rk * disp[2:3], wav * disp[3:4]], axis=0)
    emb_ref[...] = jax.lax.dot_general(
        sb, wblk_ref[...], (((0,), (0,)), ((), ())),
        preferred_element_type=jnp.float32)


def kernel(timestamp_input, auxiliary_features, Wr, br,
           W_fourier, W_spline, W_rkhs, W_wavelet):
    t_row = timestamp_input.reshape(1, B)
    auxt = auxiliary_features.T
    wrt = Wr.T
    brc = br.reshape(NE, 1)
    wblk = jnp.zeros((4 * 16, 4 * D), dtype=jnp.float32)
    for i, We in enumerate((W_fourier, W_spline, W_rkhs, W_wavelet)):
        wblk = wblk.at[16 * i:16 * (i + 1), D * i:D * (i + 1)].set(We)

    grid = (B // BLK,)
    emb, w_t, mask_t = pl.pallas_call(
        _body,
        grid=grid,
        in_specs=[
            pl.BlockSpec((1, BLK), lambda i: (0, i)),
            pl.BlockSpec((16, BLK), lambda i: (0, i)),
            pl.BlockSpec((NE, 17), lambda i: (0, 0)),
            pl.BlockSpec((NE, 1), lambda i: (0, 0)),
            pl.BlockSpec((64, 4 * D), lambda i: (0, 0)),
        ],
        out_specs=[
            pl.BlockSpec((BLK, 4 * D), lambda i: (i, 0)),
            pl.BlockSpec((NE, BLK), lambda i: (0, i)),
            pl.BlockSpec((NE, BLK), lambda i: (0, i)),
        ],
        out_shape=[
            jax.ShapeDtypeStruct((B, 4 * D), jnp.float32),
            jax.ShapeDtypeStruct((NE, B), jnp.float32),
            jax.ShapeDtypeStruct((NE, B), jnp.float32),
        ],
    )(t_row, auxt, wrt, brc, wblk)

    return emb, w_t.T, mask_t.T.astype(jnp.bool_)
